# Initial kernel scaffold; baseline (speedup 1.0000x reference)
import jax
import jax.numpy as jnp
from jax.experimental import pallas as pl


def kernel(lig_x_gen, prot_x_gen, lig_x_true, prot_x_true, lig_batch, prot_batch, time_weights):
    raise NotImplementedError("write your pallas kernel here")


# speedup vs baseline: 33.7964x; 33.7964x over previous
import functools

import jax
import jax.numpy as jnp
from jax import lax
from jax.experimental import pallas as pl
from jax.experimental.pallas import tpu as pltpu
from jax.experimental.pallas import tpu_sc as plsc

N_LIG = 2048
N_PROT = 16384
N_PROT_PAD = N_PROT + 16
N_BATCH = 32
D2_MAX = 4.5 * 4.5
K_NBR = 15
EPS = 1e-8

NW = 32
QPW = N_LIG // NW
L = 16

_INF = float("inf")


def _sqrt16(x):
    xi = plsc.bitcast(x, jnp.int32)
    y = plsc.bitcast(jnp.int32(0x5F3759DF) - (xi >> 1), jnp.float32)
    for _ in range(3):
        y = y * (1.5 - 0.5 * x * y * y)
    return x * y


def _sc_body(lgx_h, lgy_h, lgz_h, ltx_h, lty_h, ltz_h,
             pgx_h, pgy_h, pgz_h, ptx_h, pty_h, ptz_h,
             lb_hbm, pb_hbm, tw_hbm,
             se_hbm, cnt_hbm,
             lgx, lgy, lgz, ltx, lty, ltz, lb, tww, cnts,
             pgx, pgy, pgz, ptx, pty, ptz, pb, ovec):
    wid = lax.axis_index("s") * 2 + lax.axis_index("c")
    base = wid * QPW

    pltpu.sync_copy(pgx_h, pgx)
    pltpu.sync_copy(pgy_h, pgy)
    pltpu.sync_copy(pgz_h, pgz)
    pltpu.sync_copy(ptx_h, ptx)
    pltpu.sync_copy(pty_h, pty)
    pltpu.sync_copy(ptz_h, ptz)
    pltpu.sync_copy(pb_hbm, pb)
    pltpu.sync_copy(lgx_h.at[pl.ds(base, QPW)], lgx)
    pltpu.sync_copy(lgy_h.at[pl.ds(base, QPW)], lgy)
    pltpu.sync_copy(lgz_h.at[pl.ds(base, QPW)], lgz)
    pltpu.sync_copy(ltx_h.at[pl.ds(base, QPW)], ltx)
    pltpu.sync_copy(lty_h.at[pl.ds(base, QPW)], lty)
    pltpu.sync_copy(ltz_h.at[pl.ds(base, QPW)], ltz)
    pltpu.sync_copy(lb_hbm.at[pl.ds(base, QPW)], lb)
    pltpu.sync_copy(tw_hbm, tww)

    lane = lax.iota(jnp.int32, L)

    for g in range(3):
        bvec = lane + g * L
        lo = jnp.zeros((L,), jnp.int32)
        p = N_PROT // 2
        while p >= 1:
            v = plsc.load_gather(pb, [lo + (p - 1)])
            lo = jnp.where(v < bvec, lo + p, lo)
            p //= 2
        v = plsc.load_gather(pb, [lo])
        lo = jnp.where(v < bvec, lo + 1, lo)
        cnts[pl.ds(g * L, L)] = lo

    inf16 = jnp.full((L,), _INF)

    def qbody(i, carry):
        ca, na = carry
        isplat = jnp.full((L,), i, jnp.int32)
        bv = plsc.load_gather(lb, [isplat])
        sv = plsc.load_gather(cnts, [bv])
        ev = plsc.load_gather(cnts, [bv + 1])
        twv = plsc.load_gather(tww, [bv])
        ltxv = plsc.load_gather(ltx, [isplat])
        ltyv = plsc.load_gather(lty, [isplat])
        ltzv = plsc.load_gather(ltz, [isplat])
        s_sc = sv[0]
        e_sc = ev[0]
        s0 = s_sc & ~(L - 1)
        nblk = (e_sc - s0 + (L - 1)) >> 4

        def tblock(t, c2):
            av, ap = c2
            j0 = s0 + t * L
            posv = lane + j0
            dx = ptx[pl.ds(j0, L)] - ltxv
            dy = pty[pl.ds(j0, L)] - ltyv
            dz = ptz[pl.ds(j0, L)] - ltzv
            d2 = dx * dx + dy * dy + dz * dz
            inseg = (posv >= sv) & (posv < ev)
            d2 = jnp.where(inseg, d2, _INF)
            bv, bp = plsc.sort_key_val(d2, posv, descending=True)
            takeb = bv < av
            mv = jnp.where(takeb, bv, av)
            mp = jnp.where(takeb, bp, ap)
            av, ap = plsc.sort_key_val(mv, mp)
            return av, ap

        av, ap = lax.fori_loop(0, nblk, tblock, (inf16, jnp.zeros((L,), jnp.int32)))

        validm = (av <= D2_MAX) & (lane < K_NBR)
        d2t = jnp.minimum(av, 1e8)
        pgxv = plsc.load_gather(pgx, [ap])
        pgyv = plsc.load_gather(pgy, [ap])
        pgzv = plsc.load_gather(pgz, [ap])
        gx = plsc.load_gather(lgx, [isplat]) - pgxv
        gy = plsc.load_gather(lgy, [isplat]) - pgyv
        gz = plsc.load_gather(lgz, [isplat]) - pgzv
        d2g = gx * gx + gy * gy + gz * gz
        dij_g = _sqrt16(jnp.maximum(d2g, EPS))
        dij_t = _sqrt16(jnp.maximum(d2t, EPS))
        se = (dij_g - dij_t) * (dij_g - dij_t)
        ca = ca + jnp.where(validm, se * twv, 0.0)
        na = na + jnp.where(validm, 1.0, 0.0)
        return ca, na

    zero16 = jnp.zeros((L,), jnp.float32)
    ca, na = lax.fori_loop(0, QPW, qbody, (zero16, zero16))
    ovec[...] = ca
    pltpu.sync_copy(ovec, se_hbm.at[wid])
    ovec[...] = na
    pltpu.sync_copy(ovec, cnt_hbm.at[wid])


@jax.jit
def _run(lgx, lgy, lgz, ltx, lty, ltz, pgx, pgy, pgz, ptx, pty, ptz,
         lb, pb, tw):
    mesh = plsc.VectorSubcoreMesh(core_axis_name="c", subcore_axis_name="s",
                                  num_cores=2, num_subcores=16)
    f32, i32 = jnp.float32, jnp.int32
    kfn = pl.kernel(
        _sc_body,
        out_type=(
            jax.ShapeDtypeStruct((NW, L), f32),
            jax.ShapeDtypeStruct((NW, L), f32),
        ),
        mesh=mesh,
        compiler_params=pltpu.CompilerParams(needs_layout_passes=False),
        scratch_types=(
            pltpu.VMEM((QPW,), f32), pltpu.VMEM((QPW,), f32),
            pltpu.VMEM((QPW,), f32), pltpu.VMEM((QPW,), f32),
            pltpu.VMEM((QPW,), f32), pltpu.VMEM((QPW,), f32),
            pltpu.VMEM((QPW,), i32),
            pltpu.VMEM((N_BATCH,), f32),
            pltpu.VMEM((3 * L,), i32),
            pltpu.VMEM((N_PROT_PAD,), f32), pltpu.VMEM((N_PROT_PAD,), f32),
            pltpu.VMEM((N_PROT_PAD,), f32),
            pltpu.VMEM((N_PROT_PAD,), f32), pltpu.VMEM((N_PROT_PAD,), f32),
            pltpu.VMEM((N_PROT_PAD,), f32),
            pltpu.VMEM((N_PROT,), i32),
            pltpu.VMEM((L,), f32),
        ),
    )
    se, cnt = kfn(lgx, lgy, lgz, ltx, lty, ltz, pgx, pgy, pgz, ptx, pty, ptz,
                  lb, pb, tw)
    total = jnp.sum(se)
    n = jnp.sum(cnt)
    return total / jnp.maximum(n, 1.0)


def kernel(lig_x_gen, prot_x_gen, lig_x_true, prot_x_true, lig_batch,
           prot_batch, time_weights):
    pad = jnp.zeros((N_PROT_PAD - N_PROT,), jnp.float32)
    pg = [jnp.concatenate([prot_x_gen[:, c], pad]) for c in range(3)]
    pt = [jnp.concatenate([prot_x_true[:, c], pad]) for c in range(3)]
    lg = [lig_x_gen[:, c] for c in range(3)]
    lt = [lig_x_true[:, c] for c in range(3)]
    return _run(
        *lg, *lt, *pg, *pt,
        lig_batch.astype(jnp.int32), prot_batch.astype(jnp.int32),
        time_weights,
    )
